# baseline (speedup 1.0000x reference)
import functools

import jax
import jax.numpy as jnp
from jax import lax
from jax.experimental import pallas as pl
from jax.experimental.pallas import tpu as pltpu
from jax.experimental.pallas import tpu_sc as plsc

_INFO = plsc.get_sparse_core_info()
_NC = _INFO.num_cores
_NS = _INFO.num_subcores
_NW = _NC * _NS

_B = 4096 * 200
_D = 128
_V = 255
_GRP = 128
_ROWS = _B // _D
_GPW = _ROWS // _NW
_NSLOT = 2
_NR = _GPW // _NSLOT


def _body(x_hbm, tbl_hbm, out_hbm, idx_v, bufs, tbl_sh, gsem, wsem):
    sid = lax.axis_index("s")
    wid = sid * _NC + lax.axis_index("c")
    base = wid * _GPW

    @pl.when(sid == 0)
    def _():
        pltpu.sync_copy(tbl_hbm, tbl_sh)

    pltpu.sync_copy(x_hbm.at[pl.ds(base, _GPW)], idx_v)
    plsc.subcore_barrier()

    def gather(r, s, p):
        pltpu.async_copy(
            tbl_sh.at[idx_v.at[r * _NSLOT + s]], bufs.at[s, p], gsem.at[s]
        )

    def gather_wait(s, p):
        pltpu.make_async_copy(
            tbl_sh.at[idx_v.at[s]], bufs.at[s, p], gsem.at[s]
        ).wait()

    def wb(r, s, p):
        pltpu.async_copy(
            bufs.at[s, p],
            out_hbm.at[pl.ds((base + r * _NSLOT + s) * _GRP, _GRP)],
            wsem.at[s],
        )

    def wb_wait(s, p):
        pltpu.make_async_copy(
            bufs.at[s, p], out_hbm.at[pl.ds(0, _GRP)], wsem.at[s]
        ).wait()

    for s in range(_NSLOT):
        gather(0, s, 0)
    for s in range(_NSLOT):
        gather_wait(s, 0)
        wb(0, s, 0)
        gather(1, s, 1)

    def round_pair(r2, carry):
        r = 1 + 2 * r2
        for dr, p in ((0, 1), (1, 0)):
            for s in range(_NSLOT):
                wb_wait(s, 1 - p)
                gather_wait(s, p)
                wb(r + dr, s, p)
                gather(r + dr + 1, s, 1 - p)
        return carry

    lax.fori_loop(0, (_NR - 2) // 2, round_pair, 0)

    rl = _NR - 1
    pl_ = rl % 2
    for s in range(_NSLOT):
        wb_wait(s, 1 - pl_)
        gather_wait(s, pl_)
        wb(rl, s, pl_)
    for s in range(_NSLOT):
        wb_wait(s, pl_)


@jax.jit
def _lookup(x2d, weight):
    k = pl.kernel(
        _body,
        out_type=jax.ShapeDtypeStruct((_B, _D), jnp.float32),
        mesh=plsc.VectorSubcoreMesh(core_axis_name="c", subcore_axis_name="s"),
        scratch_types=[
            pltpu.VMEM((_GPW, _GRP), jnp.int32),
            pltpu.VMEM((_NSLOT, 2, _GRP, _D), jnp.float32),
            pltpu.VMEM_SHARED((_V, _D), jnp.float32),
            pltpu.SemaphoreType.DMA((_NSLOT,)),
            pltpu.SemaphoreType.DMA((_NSLOT,)),
        ],
    )
    return k(x2d, weight)


def kernel(x_ns, weight):
    n, s = x_ns.shape
    x2d = x_ns.astype(jnp.int32).reshape(_ROWS, _GRP)
    out = _lookup(x2d, weight)
    return out.reshape(n, s, _D)


# speedup vs baseline: 1.0052x; 1.0052x over previous
import functools

import jax
import jax.numpy as jnp
from jax import lax
from jax.experimental import pallas as pl
from jax.experimental.pallas import tpu as pltpu
from jax.experimental.pallas import tpu_sc as plsc

_INFO = plsc.get_sparse_core_info()
_NC = _INFO.num_cores
_NS = _INFO.num_subcores
_NW = _NC * _NS

_B = 4096 * 200
_D = 128
_V = 255
_GRP = 128
_ROWS = _B // _D
_GPW = _ROWS // _NW
_PAIRS = _GPW // 2
_NBUF = 3


def _body(x_hbm, tbl_hbm, out_hbm, idx_v, bufs, tbl_sh, gsem, wsem):
    sid = lax.axis_index("s")
    wid = sid * _NC + lax.axis_index("c")
    base = wid * _GPW

    @pl.when(sid == 0)
    def _():
        pltpu.sync_copy(tbl_hbm, tbl_sh)

    pltpu.sync_copy(x_hbm.at[pl.ds(base, _GPW)], idx_v)
    plsc.subcore_barrier()

    def g2(t, b):
        for h in range(2):
            pltpu.async_copy(
                tbl_sh.at[idx_v.at[2 * t + h]],
                bufs.at[b, pl.ds(h * _GRP, _GRP)],
                gsem.at[b],
            )

    def gwait(b):
        for h in range(2):
            pltpu.make_async_copy(
                tbl_sh.at[idx_v.at[h]],
                bufs.at[b, pl.ds(h * _GRP, _GRP)],
                gsem.at[b],
            ).wait()

    def wstart(t, b):
        pltpu.async_copy(
            bufs.at[b],
            out_hbm.at[pl.ds((base + 2 * t) * _GRP, 2 * _GRP)],
            wsem.at[b],
        )

    def wwait(b):
        pltpu.make_async_copy(
            bufs.at[b], out_hbm.at[pl.ds(0, 2 * _GRP)], wsem.at[b]
        ).wait()

    for b in range(_NBUF):
        g2(b, b)

    def outer(g, carry):
        for b in range(_NBUF):
            t = g * _NBUF + b
            gwait(b)
            wstart(t, b)
            wwait(b)
            g2(t + _NBUF, b)
        return carry

    lax.fori_loop(0, 32, outer, 0)

    for b in range(_NBUF):
        gwait(b)
        wstart(96 + b, b)
        if b == 0:
            wwait(0)
            g2(99, 0)

    gwait(0)
    wstart(99, 0)
    for b in range(_NBUF):
        wwait(b)


@jax.jit
def _lookup(x2d, weight):
    k = pl.kernel(
        _body,
        out_type=jax.ShapeDtypeStruct((_B, _D), jnp.float32),
        mesh=plsc.VectorSubcoreMesh(core_axis_name="c", subcore_axis_name="s"),
        scratch_types=[
            pltpu.VMEM((_GPW, _GRP), jnp.int32),
            pltpu.VMEM((_NBUF, 2 * _GRP, _D), jnp.float32),
            pltpu.VMEM_SHARED((_V, _D), jnp.float32),
            pltpu.SemaphoreType.DMA((_NBUF,)),
            pltpu.SemaphoreType.DMA((_NBUF,)),
        ],
    )
    return k(x2d, weight)


def kernel(x_ns, weight):
    n, s = x_ns.shape
    x2d = x_ns.astype(jnp.int32).reshape(_ROWS, _GRP)
    out = _lookup(x2d, weight)
    return out.reshape(n, s, _D)
